# baseline (speedup 1.0000x reference)
import jax
import jax.numpy as jnp
from jax import lax
from jax.experimental import pallas as pl
from jax.experimental.pallas import tpu as pltpu

EPS = 1e-5
WCHUNKS = 4


def _w_copy(hbm, vmem, sems, d, slot, midx):
    rows = hbm.shape[1]
    c = rows // WCHUNKS
    return [
        pltpu.make_async_copy(
            hbm.at[d, pl.ds(k * c, c)],
            vmem.at[slot, pl.ds(k * c, c)],
            sems.at[slot, midx, k])
        for k in range(WCHUNKS)
    ]


def _body(perm_ref, ustep_ref, first_ref, uday_ref, nuniq_ref,
          x_hbm, W1_hbm, b1_ref, W2_hbm, b2_ref, g_ref, be_ref, out_hbm,
          Xs, Ys, W1s, W2s, xsem, ysem, wsem):
    B = x_hbm.shape[0]
    nu = nuniq_ref[0]

    pltpu.make_async_copy(x_hbm.at[perm_ref[0]], Xs.at[0], xsem.at[0]).start()
    pltpu.make_async_copy(x_hbm.at[perm_ref[1]], Xs.at[1], xsem.at[1]).start()
    d0 = uday_ref[0]
    for cp in _w_copy(W1_hbm, W1s, wsem, d0, 0, 0):
        cp.start()
    for cp in _w_copy(W2_hbm, W2s, wsem, d0, 0, 1):
        cp.start()

    @pl.when(nu > 1)
    def _():
        d1 = uday_ref[1]
        for cp in _w_copy(W1_hbm, W1s, wsem, d1, 1, 0):
            cp.start()
        for cp in _w_copy(W2_hbm, W2s, wsem, d1, 1, 1):
            cp.start()

    @pl.when(nu > 2)
    def _():
        d2 = uday_ref[2]
        for cp in _w_copy(W1_hbm, W1s, wsem, d2, 2, 0):
            cp.start()
        for cp in _w_copy(W2_hbm, W2s, wsem, d2, 2, 1):
            cp.start()

    def step(s, carry):
        p = ustep_ref[s]
        slot = lax.rem(p, 4)
        xslot = lax.rem(s, 3)

        @pl.when(s + 2 < B)
        def _():
            pltpu.make_async_copy(x_hbm.at[perm_ref[s + 2]],
                                  Xs.at[lax.rem(s + 2, 3)],
                                  xsem.at[lax.rem(s + 2, 3)]).start()

        is_first = first_ref[s] == 1

        @pl.when(is_first)
        def _():
            d = uday_ref[p]
            for cp in _w_copy(W1_hbm, W1s, wsem, d, slot, 0):
                cp.wait()
            for cp in _w_copy(W2_hbm, W2s, wsem, d, slot, 1):
                cp.wait()

        @pl.when(is_first & (p + 3 < nu))
        def _():
            dn = uday_ref[p + 3]
            nslot = lax.rem(p + 3, 4)
            for cp in _w_copy(W1_hbm, W1s, wsem, dn, nslot, 0):
                cp.start()
            for cp in _w_copy(W2_hbm, W2s, wsem, dn, nslot, 1):
                cp.start()

        pltpu.make_async_copy(x_hbm.at[perm_ref[s]], Xs.at[xslot],
                              xsem.at[xslot]).wait()

        @pl.when(s >= 3)
        def _():
            pltpu.make_async_copy(Ys.at[xslot], out_hbm.at[perm_ref[s - 3]],
                                  ysem.at[xslot]).wait()

        d = uday_ref[p]
        xb = Xs[xslot].astype(jnp.bfloat16)
        h = jnp.dot(xb, W1s[slot].astype(jnp.bfloat16),
                    preferred_element_type=jnp.float32)
        h = jnp.maximum(h + b1_ref[d], 0.0).astype(jnp.bfloat16)
        y = jnp.dot(h, W2s[slot].astype(jnp.bfloat16),
                    preferred_element_type=jnp.float32)
        y = y + b2_ref[d]
        mu = jnp.mean(y, axis=-1, keepdims=True)
        yc = y - mu
        var = jnp.mean(yc * yc, axis=-1, keepdims=True)
        Ys[xslot] = yc * lax.rsqrt(var + EPS) * g_ref[d] + be_ref[d]

        pltpu.make_async_copy(Ys.at[xslot], out_hbm.at[perm_ref[s]],
                              ysem.at[xslot]).start()
        return carry

    lax.fori_loop(0, B, step, 0, unroll=False)

    for k in range(3):
        s = B - 3 + k
        pltpu.make_async_copy(Ys.at[lax.rem(s, 3)],
                              out_hbm.at[perm_ref[s]],
                              ysem.at[lax.rem(s, 3)]).wait()


def kernel(x, day_indicies, W1, b1, W2, b2, gamma, beta):
    B, T, IN = x.shape
    D, _, HID = W1.shape
    OUT = W2.shape[2]

    day = day_indicies.astype(jnp.int32)
    perm = jnp.argsort(day).astype(jnp.int32)
    sdays = jnp.take(day, perm)

    first = jnp.concatenate(
        [jnp.ones((1,), jnp.int32),
         (sdays[1:] != sdays[:-1]).astype(jnp.int32)])
    ustep = jnp.cumsum(first) - 1
    uday = jnp.zeros((B,), jnp.int32).at[ustep].set(sdays)
    nuniq = jnp.sum(first).reshape(1)

    b1r = b1.reshape(D, 1, HID)
    b2r = b2.reshape(D, 1, OUT)
    gr = gamma.reshape(D, 1, OUT)
    br = beta.reshape(D, 1, OUT)

    vec_spec = pl.BlockSpec(memory_space=pltpu.MemorySpace.VMEM)
    hbm = pl.BlockSpec(memory_space=pltpu.MemorySpace.HBM)

    grid_spec = pltpu.PrefetchScalarGridSpec(
        num_scalar_prefetch=5,
        grid=(1,),
        in_specs=[hbm, hbm, vec_spec, hbm, vec_spec, vec_spec, vec_spec],
        out_specs=hbm,
        scratch_shapes=[
            pltpu.VMEM((3, T, IN), jnp.float32),
            pltpu.VMEM((3, T, OUT), jnp.float32),
            pltpu.VMEM((4, IN, HID), jnp.float32),
            pltpu.VMEM((4, HID, OUT), jnp.float32),
            pltpu.SemaphoreType.DMA((3,)),
            pltpu.SemaphoreType.DMA((3,)),
            pltpu.SemaphoreType.DMA((4, 2, WCHUNKS)),
        ],
    )

    return pl.pallas_call(
        _body,
        grid_spec=grid_spec,
        out_shape=jax.ShapeDtypeStruct((B, T, OUT), jnp.float32),
        compiler_params=pltpu.CompilerParams(
            dimension_semantics=("arbitrary",),
        ),
    )(perm, ustep, first, uday, nuniq,
      x, W1, b1r, W2, b2r, gr, br)


# speedup vs baseline: 1.0195x; 1.0195x over previous
import jax
import jax.numpy as jnp
from jax import lax
from jax.experimental import pallas as pl
from jax.experimental.pallas import tpu as pltpu

EPS = 1e-5
WCHUNKS = 4


def _w_copy(hbm, vmem, sems, d, slot, midx):
    rows = hbm.shape[1]
    c = rows // WCHUNKS
    return [
        pltpu.make_async_copy(
            hbm.at[d, pl.ds(k * c, c)],
            vmem.at[slot, pl.ds(k * c, c)],
            sems.at[slot, midx, k])
        for k in range(WCHUNKS)
    ]


def _body(perm_ref, ustep_ref, first_ref, uday_ref, nuniq_ref,
          x_hbm, W1_hbm, b1_ref, W2_hbm, b2_ref, g_ref, be_ref, out_hbm,
          Xs, Ys, W1s, W2s, xsem, ysem, wsem):
    B = x_hbm.shape[0]
    nu = nuniq_ref[0]

    pltpu.make_async_copy(x_hbm.at[perm_ref[0]], Xs.at[0], xsem.at[0]).start()
    pltpu.make_async_copy(x_hbm.at[perm_ref[1]], Xs.at[1], xsem.at[1]).start()
    d0 = uday_ref[0]
    for cp in _w_copy(W1_hbm, W1s, wsem, d0, 0, 0):
        cp.start()
    for cp in _w_copy(W2_hbm, W2s, wsem, d0, 0, 1):
        cp.start()

    @pl.when(nu > 1)
    def _():
        d1 = uday_ref[1]
        for cp in _w_copy(W1_hbm, W1s, wsem, d1, 1, 0):
            cp.start()
        for cp in _w_copy(W2_hbm, W2s, wsem, d1, 1, 1):
            cp.start()

    def step(s, carry):
        p = ustep_ref[s]
        slot = lax.rem(p, 3)
        xslot = lax.rem(s, 3)

        @pl.when(s + 2 < B)
        def _():
            pltpu.make_async_copy(x_hbm.at[perm_ref[s + 2]],
                                  Xs.at[lax.rem(s + 2, 3)],
                                  xsem.at[lax.rem(s + 2, 3)]).start()

        is_first = first_ref[s] == 1

        @pl.when(is_first)
        def _():
            d = uday_ref[p]
            for cp in _w_copy(W1_hbm, W1s, wsem, d, slot, 0):
                cp.wait()
            for cp in _w_copy(W2_hbm, W2s, wsem, d, slot, 1):
                cp.wait()

        @pl.when(is_first & (p + 2 < nu))
        def _():
            dn = uday_ref[p + 2]
            nslot = lax.rem(p + 2, 3)
            for cp in _w_copy(W1_hbm, W1s, wsem, dn, nslot, 0):
                cp.start()
            for cp in _w_copy(W2_hbm, W2s, wsem, dn, nslot, 1):
                cp.start()

        pltpu.make_async_copy(x_hbm.at[perm_ref[s]], Xs.at[xslot],
                              xsem.at[xslot]).wait()

        @pl.when(s >= 3)
        def _():
            pltpu.make_async_copy(Ys.at[xslot], out_hbm.at[perm_ref[s - 3]],
                                  ysem.at[xslot]).wait()

        d = uday_ref[p]
        xb = Xs[xslot].astype(jnp.bfloat16)
        h = jnp.dot(xb, W1s[slot].astype(jnp.bfloat16),
                    preferred_element_type=jnp.float32)
        h = jnp.maximum(h + b1_ref[d], 0.0).astype(jnp.bfloat16)
        y = jnp.dot(h, W2s[slot].astype(jnp.bfloat16),
                    preferred_element_type=jnp.float32)
        y = y + b2_ref[d]
        mu = jnp.mean(y, axis=-1, keepdims=True)
        yc = y - mu
        var = jnp.mean(yc * yc, axis=-1, keepdims=True)
        Ys[xslot] = yc * lax.rsqrt(var + EPS) * g_ref[d] + be_ref[d]

        pltpu.make_async_copy(Ys.at[xslot], out_hbm.at[perm_ref[s]],
                              ysem.at[xslot]).start()
        return carry

    lax.fori_loop(0, B, step, 0, unroll=False)

    for k in range(3):
        s = B - 3 + k
        pltpu.make_async_copy(Ys.at[lax.rem(s, 3)],
                              out_hbm.at[perm_ref[s]],
                              ysem.at[lax.rem(s, 3)]).wait()


def kernel(x, day_indicies, W1, b1, W2, b2, gamma, beta):
    B, T, IN = x.shape
    D, _, HID = W1.shape
    OUT = W2.shape[2]

    day = day_indicies.astype(jnp.int32)
    perm = jnp.argsort(day).astype(jnp.int32)
    sdays = jnp.take(day, perm)

    first = jnp.concatenate(
        [jnp.ones((1,), jnp.int32),
         (sdays[1:] != sdays[:-1]).astype(jnp.int32)])
    ustep = jnp.cumsum(first) - 1
    uday = jnp.zeros((B,), jnp.int32).at[ustep].set(sdays)
    nuniq = jnp.sum(first).reshape(1)

    b1r = b1.reshape(D, 1, HID)
    b2r = b2.reshape(D, 1, OUT)
    gr = gamma.reshape(D, 1, OUT)
    br = beta.reshape(D, 1, OUT)

    vec_spec = pl.BlockSpec(memory_space=pltpu.MemorySpace.VMEM)
    hbm = pl.BlockSpec(memory_space=pltpu.MemorySpace.HBM)

    grid_spec = pltpu.PrefetchScalarGridSpec(
        num_scalar_prefetch=5,
        grid=(1,),
        in_specs=[hbm, hbm, vec_spec, hbm, vec_spec, vec_spec, vec_spec],
        out_specs=hbm,
        scratch_shapes=[
            pltpu.VMEM((3, T, IN), jnp.float32),
            pltpu.VMEM((3, T, OUT), jnp.float32),
            pltpu.VMEM((3, IN, HID), jnp.float32),
            pltpu.VMEM((3, HID, OUT), jnp.float32),
            pltpu.SemaphoreType.DMA((3,)),
            pltpu.SemaphoreType.DMA((3,)),
            pltpu.SemaphoreType.DMA((3, 2, WCHUNKS)),
        ],
    )

    return pl.pallas_call(
        _body,
        grid_spec=grid_spec,
        out_shape=jax.ShapeDtypeStruct((B, T, OUT), jnp.float32),
        compiler_params=pltpu.CompilerParams(
            dimension_semantics=("arbitrary",),
        ),
    )(perm, ustep, first, uday, nuniq,
      x, W1, b1r, W2, b2r, gr, br)
